# baseline (speedup 1.0000x reference)
import jax
import jax.numpy as jnp
from jax.experimental import pallas as pl

_EPS = 1e-12
_SEQ_BLOCK = 256


def _ln_kernel(we_ref, pos_ref, w_ref, b_ref, out_ref):
    pos = pos_ref[...]
    w = w_ref[...]
    b = b_ref[...]
    x = we_ref[...] + pos[None, :, :]
    mean = jnp.mean(x, axis=-1, keepdims=True)
    xc = x - mean
    var = jnp.mean(xc * xc, axis=-1, keepdims=True)
    out_ref[...] = xc * (jax.lax.rsqrt(var + _EPS) * w) + b


def kernel(word_embeddings, pos_table, ln_weight, ln_bias):
    batch, seq, hidden = word_embeddings.shape
    s = _SEQ_BLOCK
    grid = (seq // s,)
    return pl.pallas_call(
        _ln_kernel,
        grid=grid,
        in_specs=[
            pl.BlockSpec((batch, s, hidden), lambda i: (0, i, 0)),
            pl.BlockSpec((s, hidden), lambda i: (i, 0)),
            pl.BlockSpec((hidden,), lambda i: (0,)),
            pl.BlockSpec((hidden,), lambda i: (0,)),
        ],
        out_specs=pl.BlockSpec((batch, s, hidden), lambda i: (0, i, 0)),
        out_shape=jax.ShapeDtypeStruct((batch, seq, hidden), jnp.float32),
    )(word_embeddings, pos_table[:seq], ln_weight, ln_bias)


# speedup vs baseline: 1.0414x; 1.0414x over previous
import jax
import jax.numpy as jnp
from jax.experimental import pallas as pl

_EPS = 1e-12
_SEQ_BLOCK = 512
_ROW_CHUNK = 64


def _ln_kernel(we_ref, pos_ref, w_ref, b_ref, out_ref):
    w = w_ref[...]
    b = b_ref[...]
    batch, s, _ = we_ref.shape

    def body(i, _):
        r = i * _ROW_CHUNK
        pos = pos_ref[pl.ds(r, _ROW_CHUNK), :]
        for bi in range(batch):
            x = we_ref[bi, pl.ds(r, _ROW_CHUNK), :] + pos
            mean = jnp.mean(x, axis=-1, keepdims=True)
            xc = x - mean
            var = jnp.mean(xc * xc, axis=-1, keepdims=True)
            out_ref[bi, pl.ds(r, _ROW_CHUNK), :] = (
                xc * (jax.lax.rsqrt(var + _EPS) * w) + b)
        return 0

    jax.lax.fori_loop(0, s // _ROW_CHUNK, body, 0)


def kernel(word_embeddings, pos_table, ln_weight, ln_bias):
    batch, seq, hidden = word_embeddings.shape
    s = _SEQ_BLOCK
    grid = (seq // s,)
    return pl.pallas_call(
        _ln_kernel,
        grid=grid,
        in_specs=[
            pl.BlockSpec((batch, s, hidden), lambda i: (0, i, 0)),
            pl.BlockSpec((s, hidden), lambda i: (i, 0)),
            pl.BlockSpec((hidden,), lambda i: (0,)),
            pl.BlockSpec((hidden,), lambda i: (0,)),
        ],
        out_specs=pl.BlockSpec((batch, s, hidden), lambda i: (0, i, 0)),
        out_shape=jax.ShapeDtypeStruct((batch, seq, hidden), jnp.float32),
    )(word_embeddings, pos_table[:seq], ln_weight, ln_bias)


# speedup vs baseline: 1.1072x; 1.0632x over previous
import jax
import jax.numpy as jnp
from jax.experimental import pallas as pl

_EPS = 1e-12
_SEQ_BLOCK = 768
_ROW_CHUNK = 64


def _ln_kernel(we_ref, pos_ref, w_ref, b_ref, out_ref):
    w = w_ref[...]
    b = b_ref[...]
    batch, s, _ = we_ref.shape

    def body(i, _):
        r = i * _ROW_CHUNK
        pos = pos_ref[pl.ds(r, _ROW_CHUNK), :]
        for bi in range(batch):
            x = we_ref[bi, pl.ds(r, _ROW_CHUNK), :] + pos
            mean = jnp.mean(x, axis=-1, keepdims=True)
            xc = x - mean
            var = jnp.mean(xc * xc, axis=-1, keepdims=True)
            out_ref[bi, pl.ds(r, _ROW_CHUNK), :] = (
                xc * (jax.lax.rsqrt(var + _EPS) * w) + b)
        return 0

    jax.lax.fori_loop(0, s // _ROW_CHUNK, body, 0)


def kernel(word_embeddings, pos_table, ln_weight, ln_bias):
    batch, seq, hidden = word_embeddings.shape
    s = _SEQ_BLOCK
    grid = (seq // s,)
    return pl.pallas_call(
        _ln_kernel,
        grid=grid,
        in_specs=[
            pl.BlockSpec((batch, s, hidden), lambda i: (0, i, 0)),
            pl.BlockSpec((s, hidden), lambda i: (i, 0)),
            pl.BlockSpec((hidden,), lambda i: (0,)),
            pl.BlockSpec((hidden,), lambda i: (0,)),
        ],
        out_specs=pl.BlockSpec((batch, s, hidden), lambda i: (0, i, 0)),
        out_shape=jax.ShapeDtypeStruct((batch, seq, hidden), jnp.float32),
    )(word_embeddings, pos_table[:seq], ln_weight, ln_bias)
